# baseline (speedup 1.0000x reference)
import functools

import jax
import jax.numpy as jnp
from jax import lax
from jax.experimental import pallas as pl
from jax.experimental.pallas import tpu as pltpu
from jax.experimental.pallas import tpu_sc as plsc

_COMMITMENT_WEIGHT = 0.25
_BLK = 256
_NC = 2
_NS = 16
_RPW = 8192 // (_NC * _NS)


def _sc_loss_body(z_hbm, e_hbm, idx_hbm, out_hbm, idx_v, q_v, z_v, acc_v, sem):
    wid = lax.axis_index("s") * _NC + lax.axis_index("c")
    base = wid * _RPW
    pltpu.sync_copy(idx_hbm.at[pl.ds(base, _RPW)], idx_v)
    pltpu.async_copy(e_hbm.at[idx_v], q_v, sem).wait()
    pltpu.sync_copy(z_hbm.at[pl.ds(base, _RPW)], z_v)

    def body(r, acc):
        d0 = z_v[r, 0:16] - q_v[r, 0:16]
        d1 = z_v[r, 16:32] - q_v[r, 16:32]
        return acc + d0 * d0 + d1 * d1

    acc = lax.fori_loop(0, _RPW, body, jnp.zeros((16,), jnp.float32))
    acc_v[...] = acc
    pltpu.sync_copy(acc_v, out_hbm.at[wid])


def _sc_loss_partials(z, embedding, idx):
    k = functools.partial(
        pl.kernel,
        out_type=jax.ShapeDtypeStruct((_NC * _NS, 16), jnp.float32),
        mesh=plsc.VectorSubcoreMesh(core_axis_name="c", subcore_axis_name="s"),
        scratch_types=[
            pltpu.VMEM((_RPW,), jnp.int32),
            pltpu.VMEM((_RPW, 128), jnp.float32),
            pltpu.VMEM((_RPW, 32), jnp.float32),
            pltpu.VMEM((16,), jnp.float32),
            pltpu.SemaphoreType.DMA,
        ],
    )(_sc_loss_body)
    e_pad = jnp.pad(embedding, ((0, 0), (0, 96)))
    return k(z, e_pad, idx)


def _vq_body(z_ref, e_ref, zsq_ref, esq_ref, idx_ref, msum_ref):
    z = z_ref[...].astype(jnp.bfloat16).astype(jnp.float32)
    e = e_ref[...].astype(jnp.bfloat16).astype(jnp.float32)
    ze2 = 2.0 * lax.dot_general(
        z, e, (((1,), (1,)), ((), ())), preferred_element_type=jnp.float32)
    d = (zsq_ref[...] - ze2) + esq_ref[...]
    m = jnp.min(d, axis=1, keepdims=True)
    iota = lax.broadcasted_iota(jnp.int32, d.shape, 1)
    idx = jnp.min(jnp.where(d <= m, iota, jnp.int32(2 ** 30)),
                  axis=1, keepdims=True)
    idx_ref[...] = idx
    msum_ref[...] = jnp.sum(m).reshape(1, 1, 1)


def kernel(encoded_latents, embedding):
    encoded_latents = encoded_latents.astype(jnp.float32)
    B, N, D = encoded_latents.shape
    CB = embedding.shape[0]
    rows = B * N
    grid = rows // _BLK
    z = encoded_latents.reshape(rows, D)
    zsq = jnp.sum(z ** 2, axis=1, keepdims=True)
    esq = jnp.sum(embedding ** 2, axis=1)[None, :]

    idx2d, msum = pl.pallas_call(
        _vq_body,
        grid=(grid,),
        in_specs=[
            pl.BlockSpec((_BLK, D), lambda i: (i, 0)),
            pl.BlockSpec((CB, D), lambda i: (0, 0)),
            pl.BlockSpec((_BLK, 1), lambda i: (i, 0)),
            pl.BlockSpec((1, CB), lambda i: (0, 0)),
        ],
        out_specs=[
            pl.BlockSpec((_BLK, 1), lambda i: (i, 0)),
            pl.BlockSpec((1, 1, 1), lambda i: (i, 0, 0)),
        ],
        out_shape=[
            jax.ShapeDtypeStruct((rows, 1), jnp.int32),
            jax.ShapeDtypeStruct((grid, 1, 1), jnp.float32),
        ],
    )(z, embedding, zsq, esq)

    quantized_indices = idx2d.reshape(B, N)
    del msum
    partials = _sc_loss_partials(z, embedding, idx2d.reshape(rows))
    commitment_loss = _COMMITMENT_WEIGHT * (jnp.sum(partials) / (rows * D))
    return (encoded_latents, quantized_indices, commitment_loss)
